# Initial kernel scaffold; baseline (speedup 1.0000x reference)
import jax
import jax.numpy as jnp
from jax.experimental import pallas as pl


def kernel(achieved_goal, desired_goal):
    raise NotImplementedError("write your pallas kernel here")


# speedup vs baseline: 1.9832x; 1.9832x over previous
import jax
import jax.numpy as jnp
from jax.experimental import pallas as pl

_BS, _NV, _NP, _FD = 64, 4, 512, 10
_THR = 6.0
_SCALE = 1.0


def _chamfer_body(goal_ref, stateT_ref, out_ref):
    iota_l = jax.lax.broadcasted_iota(jnp.int32, (_NP, _NP), 1)
    iota_s = jax.lax.broadcasted_iota(jnp.int32, (_NP, _NP), 0)
    total = None
    for v in range(_NV):
        g = goal_ref[0, v]
        sT = stateT_ref[0, v]

        P = None
        for f in range(5, 9):
            d = g[:, f:f + 1] - sT[f:f + 1, :]
            P = d * d if P is None else P + d * d

        minv_g = jnp.min(P, axis=1, keepdims=True)
        idx_g = jnp.min(jnp.where(P == minv_g, iota_l, _NP),
                        axis=1, keepdims=True)
        sel = iota_l == idx_g
        sx = jnp.sum(jnp.where(sel, sT[0:1, :], 0.0), axis=1, keepdims=True)
        sy = jnp.sum(jnp.where(sel, sT[1:2, :], 0.0), axis=1, keepdims=True)
        dx = g[:, 0:1] - sx
        dy = g[:, 1:2] - sy
        xy1 = jnp.sqrt(dx * dx + dy * dy)
        xy1 = jnp.where(minv_g > _THR, 1.0, xy1)
        s1 = jnp.sum(xy1)

        minv_s = jnp.min(P, axis=0, keepdims=True)
        idx_s = jnp.min(jnp.where(P == minv_s, iota_s, _NP),
                        axis=0, keepdims=True)
        sel2 = iota_s == idx_s
        gx = jnp.sum(jnp.where(sel2, g[:, 0:1], 0.0), axis=0, keepdims=True)
        gy = jnp.sum(jnp.where(sel2, g[:, 1:2], 0.0), axis=0, keepdims=True)
        dx2 = sT[0:1, :] - gx
        dy2 = sT[1:2, :] - gy
        xy2 = jnp.sqrt(dx2 * dx2 + dy2 * dy2)
        xy2 = jnp.where(minv_s > _THR, 1.0, xy2)
        s2 = jnp.sum(xy2)

        part = s1 + s2
        total = part if total is None else total + part

    out_ref[...] = (total * (-_SCALE / (2.0 * _NP * _NV))).reshape(1, 1, 1)


@jax.jit
def kernel(achieved_goal, desired_goal):
    stateT = jnp.swapaxes(achieved_goal, -1, -2)
    out = pl.pallas_call(
        _chamfer_body,
        grid=(_BS,),
        in_specs=[
            pl.BlockSpec((1, _NV, _NP, _FD), lambda b: (b, 0, 0, 0)),
            pl.BlockSpec((1, _NV, _FD, _NP), lambda b: (b, 0, 0, 0)),
        ],
        out_specs=pl.BlockSpec((1, 1, 1), lambda b: (b, 0, 0)),
        out_shape=jax.ShapeDtypeStruct((_BS, 1, 1), jnp.float32),
    )(desired_goal, stateT)
    return out.reshape(_BS, 1)
